# Initial kernel scaffold; baseline (speedup 1.0000x reference)
import jax
import jax.numpy as jnp
from jax.experimental import pallas as pl


def kernel(table):
    raise NotImplementedError("write your pallas kernel here")


# speedup vs baseline: 9.7913x; 9.7913x over previous
import functools

import jax
import jax.numpy as jnp
from jax import lax
from jax.experimental import pallas as pl
from jax.experimental.pallas import tpu as pltpu
from jax.experimental.pallas import tpu_sc as plsc

_T = 2048
_S = 2048
_E = 16
_FLAT = (_T + _S - 1) * _E
_ROW = _S * _E

_NC = 2
_NS = 16
_NW = _NC * _NS
_ROWS_PER_W = _T // _NW

_mesh = plsc.VectorSubcoreMesh(core_axis_name="c", subcore_axis_name="s")


@functools.partial(
    pl.kernel,
    mesh=_mesh,
    out_type=jax.ShapeDtypeStruct((_T * _ROW,), jnp.float32),
    scratch_types=[pltpu.VMEM((_FLAT,), jnp.float32)],
)
def _sc_window_copy(table_hbm, out_hbm, table_v):
    wid = lax.axis_index("s") * _NC + lax.axis_index("c")
    pltpu.sync_copy(table_hbm, table_v)
    base = wid * _ROWS_PER_W

    def body(r, carry):
        i = base + r
        src = (_T - 1 - i) * _E
        pltpu.sync_copy(
            table_v.at[pl.ds(src, _ROW)],
            out_hbm.at[pl.ds(i * _ROW, _ROW)],
        )
        return carry

    lax.fori_loop(0, _ROWS_PER_W, body, 0)


def kernel(table):
    flat = table.reshape(-1)
    out = _sc_window_copy(flat)
    return out.reshape(_T, _S, _E)


# speedup vs baseline: 9.7994x; 1.0008x over previous
import functools

import jax
import jax.numpy as jnp
from jax import lax
from jax.experimental import pallas as pl
from jax.experimental.pallas import tpu as pltpu
from jax.experimental.pallas import tpu_sc as plsc

_T = 2048
_S = 2048
_E = 16
_FLAT = (_T + _S - 1) * _E
_ROW = _S * _E

_NC = 2
_NS = 16
_NW = _NC * _NS
_ROWS_PER_W = _T // _NW

_mesh = plsc.VectorSubcoreMesh(core_axis_name="c", subcore_axis_name="s")


_K = 8


@functools.partial(
    pl.kernel,
    mesh=_mesh,
    out_type=jax.ShapeDtypeStruct((_T * _ROW,), jnp.float32),
    scratch_types=[
        pltpu.VMEM((_FLAT,), jnp.float32),
        pltpu.SemaphoreType.DMA,
    ],
)
def _sc_window_copy(table_hbm, out_hbm, table_v, sem):
    wid = lax.axis_index("s") * _NC + lax.axis_index("c")
    pltpu.sync_copy(table_hbm, table_v)
    base = wid * _ROWS_PER_W

    def body(g, carry):
        i0 = base + g * _K
        descs = []
        for r in range(_K):
            i = i0 + r
            src = (_T - 1 - i) * _E
            d = pltpu.make_async_copy(
                table_v.at[pl.ds(src, _ROW)],
                out_hbm.at[pl.ds(i * _ROW, _ROW)],
                sem,
            )
            d.start()
            descs.append(d)
        for d in descs:
            d.wait()
        return carry

    lax.fori_loop(0, _ROWS_PER_W // _K, body, 0)


def kernel(table):
    flat = table.reshape(-1)
    out = _sc_window_copy(flat)
    return out.reshape(_T, _S, _E)
